# Initial kernel scaffold; baseline (speedup 1.0000x reference)
import jax
import jax.numpy as jnp
from jax.experimental import pallas as pl


def kernel(x, edge_index, W1, b1, W2, b2):
    raise NotImplementedError("write your pallas kernel here")


# speedup vs baseline: 15.4533x; 15.4533x over previous
import functools

import jax
import jax.numpy as jnp
from jax import lax
from jax.experimental import pallas as pl
from jax.experimental.pallas import tpu as pltpu
from jax.experimental.pallas import tpu_sc as plsc

N_NODES = 10000
N_EDGES = 160000
IN_DIM = 256
HID_DIM = 512

NC = 2
NS = 16
NW = NC * NS
N_PAD = 10240
RPT = N_PAD // NS

E_PAD = 160256
EPW = E_PAD // NW
EVECS = EPW // 16

HALF = IN_DIM // 2

_mesh = plsc.VectorSubcoreMesh(core_axis_name="c", subcore_axis_name="s")


@functools.partial(
    pl.kernel,
    out_type=jax.ShapeDtypeStruct((NW * N_PAD,), jnp.float32),
    mesh=_mesh,
    scratch_types=[
        pltpu.VMEM((EPW,), jnp.int32),
        pltpu.VMEM((N_PAD,), jnp.float32),
    ],
    compiler_params=pltpu.CompilerParams(needs_layout_passes=False),
)
def _sc_count(dst_flat, out, dst_v, acc_v):
    c = lax.axis_index("c")
    s = lax.axis_index("s")
    w = c * NS + s
    pltpu.sync_copy(dst_flat.at[pl.ds(w * EPW, EPW)], dst_v)

    zeros = jnp.zeros((16,), jnp.float32)
    ones = jnp.ones((16,), jnp.float32)

    @pl.loop(0, N_PAD // 16)
    def _(j):
        acc_v[pl.ds(j * 16, 16)] = zeros

    @pl.loop(0, EVECS)
    def _(i):
        d = dst_v[pl.ds(i * 16, 16)]
        plsc.addupdate_scatter(acc_v, [d], ones)

    pltpu.sync_copy(acc_v, out.at[pl.ds(w * N_PAD, N_PAD)])


def _tc_dinv_body(cnt_ref, dinv_ref):
    deg = jnp.sum(cnt_ref[...], axis=0) + 1.0
    dinv_ref[...] = lax.rsqrt(deg)


def _tc_dinv(cntp):
    blk = 16
    return pl.pallas_call(
        _tc_dinv_body,
        grid=(N_PAD // 128 // blk,),
        in_specs=[pl.BlockSpec((NW, blk, 128), lambda i: (0, i, 0))],
        out_specs=pl.BlockSpec((blk, 128), lambda i: (i, 0)),
        out_shape=jax.ShapeDtypeStruct((N_PAD // 128, 128), jnp.float32),
    )(cntp)


def _tc_prep_body(dinv_col_ref, x_ref, dinv_ref, y0_ref, y1_ref):
    dinv = dinv_col_ref[...]
    y = x_ref[...] * dinv
    y0_ref[...] = y[:, :HALF]
    y1_ref[...] = y[:, HALF:]
    dinv_ref[...] = jnp.broadcast_to(dinv, dinv_ref.shape)


def _tc_prep(dinv_col, x):
    blk = 1000
    yspec = pl.BlockSpec((blk, HALF), lambda i: (i, 0))
    yshape = jax.ShapeDtypeStruct((N_NODES, HALF), jnp.float32)
    return pl.pallas_call(
        _tc_prep_body,
        grid=(N_NODES // blk,),
        in_specs=[
            pl.BlockSpec((blk, 1), lambda i: (i, 0)),
            pl.BlockSpec((blk, IN_DIM), lambda i: (i, 0)),
        ],
        out_specs=[pl.BlockSpec((blk, 16), lambda i: (i, 0)), yspec, yspec],
        out_shape=[jax.ShapeDtypeStruct((N_NODES, 16), jnp.float32),
                   yshape, yshape],
    )(dinv_col, x)


NHALF = N_PAD // 2
RPC = NHALF // NS
EPT = N_EDGES // NS
CVECS = EPT // 16
RB = 96
CCAP = EPT + 112


@functools.partial(
    pl.kernel,
    out_type=[jax.ShapeDtypeStruct((N_PAD, HALF), jnp.float32)] * 2,
    mesh=_mesh,
    scratch_types=[
        pltpu.VMEM((EPT,), jnp.int32),
        pltpu.VMEM((EPT,), jnp.int32),
        pltpu.VMEM((CCAP,), jnp.int32),
        pltpu.VMEM((CCAP,), jnp.int32),
        pltpu.VMEM((RB,), jnp.int32),
        pltpu.VMEM((RB,), jnp.int32),
        pltpu.VMEM((RB, HALF), jnp.float32),
        pltpu.VMEM_SHARED((NHALF + 8, HALF), jnp.float32),
        pltpu.SemaphoreType.DMA,
    ],
    compiler_params=pltpu.CompilerParams(needs_layout_passes=False),
)
def _sc_rowprop(src_h, dst_h, y0, y1, z320, out0, out1,
                src_v, dst_v, src_c, dst_c, src_st, dst_st, gbuf, acc, sem):
    c = lax.axis_index("c")
    s = lax.axis_index("s")
    lo = c * NHALF
    pltpu.sync_copy(src_h.at[pl.ds(s * EPT, EPT)], src_v)
    pltpu.sync_copy(dst_h.at[pl.ds(s * EPT, EPT)], dst_v)

    @pl.loop(0, CVECS, init_carry=jnp.int32(0))
    def cnt(i, n):
        sl = pl.ds(i * 16, 16)
        sv = src_v[sl]
        dl = dst_v[sl] - lo
        m = (dl >= 0) & (dl < NHALF)
        plsc.store_compressed(src_c.at[pl.ds(n, 16)], sv, mask=m)
        plsc.store_compressed(dst_c.at[pl.ds(n, 16)], dl, mask=m)
        return n + jnp.sum(m.astype(jnp.int32))

    sent_src = jnp.zeros((16,), jnp.int32)
    sent_dst = jnp.full((16,), NHALF, jnp.int32)

    @pl.loop(0, RB // 16)
    def _(k):
        src_c[pl.ds(cnt + k * 16, 16)] = sent_src
        dst_c[pl.ds(cnt + k * 16, 16)] = sent_dst

    nb = (cnt + (RB - 1)) // RB

    def run(tbl, out):
        @pl.loop(0, nb)
        def _(k):
            @pl.loop(0, RB // 16)
            def _(j):
                jl = pl.ds(j * 16, 16)
                src_st[jl] = src_c[pl.ds(k * RB + j * 16, 16)]
                dst_st[jl] = dst_c[pl.ds(k * RB + j * 16, 16)]

            pltpu.async_copy(tbl.at[src_st], gbuf, sem).wait()
            pltpu.sync_copy(gbuf, acc.at[dst_st], add=True)

        plsc.subcore_barrier()
        pltpu.sync_copy(acc.at[pl.ds(s * RPC, RPC)],
                        out.at[pl.ds(lo + s * RPC, RPC)])

    pltpu.sync_copy(z320, acc.at[pl.ds(s * RPC, RPC)])
    plsc.subcore_barrier()
    run(y0, out0)
    pltpu.sync_copy(z320, acc.at[pl.ds(s * RPC, RPC)])
    plsc.subcore_barrier()
    run(y1, out1)


def _tc_mm_body(a0_ref, a1_ref, y0_ref, y1_ref, dinv_ref, w1_ref, b1_ref,
                w2_ref, u_ref):
    d = dinv_ref[:, 0:1]
    p0 = d * (a0_ref[...] + y0_ref[...])
    p1 = d * (a1_ref[...] + y1_ref[...])
    h = jnp.dot(p0, w1_ref[:HALF, :], preferred_element_type=jnp.float32)
    h = h + jnp.dot(p1, w1_ref[HALF:, :], preferred_element_type=jnp.float32)
    h = jnp.maximum(h + b1_ref[...], 0.0)
    z = jnp.dot(h, w2_ref[...], preferred_element_type=jnp.float32)
    u_ref[...] = jnp.broadcast_to(z * d, u_ref.shape)


def _tc_mm(acc0, acc1, y0, y1, dinv16, W1, b1r, W2):
    blk = 400
    hspec = pl.BlockSpec((blk, HALF), lambda i: (i, 0))
    return pl.pallas_call(
        _tc_mm_body,
        grid=(N_NODES // blk,),
        in_specs=[hspec, hspec, hspec, hspec,
                  pl.BlockSpec((blk, 16), lambda i: (i, 0)),
                  pl.BlockSpec((IN_DIM, HID_DIM), lambda i: (0, 0)),
                  pl.BlockSpec((1, HID_DIM), lambda i: (0, 0)),
                  pl.BlockSpec((HID_DIM, 1), lambda i: (0, 0))],
        out_specs=pl.BlockSpec((blk, 16), lambda i: (i, 0)),
        out_shape=jax.ShapeDtypeStruct((N_NODES, 16), jnp.float32),
    )(acc0, acc1, y0, y1, dinv16, W1, b1r, W2)


@functools.partial(
    pl.kernel,
    out_type=jax.ShapeDtypeStruct((NW * N_PAD,), jnp.float32),
    mesh=_mesh,
    scratch_types=[
        pltpu.VMEM((EPW,), jnp.int32),
        pltpu.VMEM((EPW,), jnp.int32),
        pltpu.VMEM((N_PAD,), jnp.float32),
        pltpu.VMEM((N_PAD,), jnp.float32),
    ],
    compiler_params=pltpu.CompilerParams(needs_layout_passes=False),
)
def _sc_scalarprop(src_flat, dst_flat, u_flat, out, src_v, dst_v, u_v, acc_v):
    c = lax.axis_index("c")
    s = lax.axis_index("s")
    w = c * NS + s
    pltpu.sync_copy(src_flat.at[pl.ds(w * EPW, EPW)], src_v)
    pltpu.sync_copy(dst_flat.at[pl.ds(w * EPW, EPW)], dst_v)
    pltpu.sync_copy(u_flat, u_v)

    zeros = jnp.zeros((16,), jnp.float32)

    @pl.loop(0, N_PAD // 16)
    def _(j):
        acc_v[pl.ds(j * 16, 16)] = zeros

    @pl.loop(0, EVECS)
    def _(i):
        sl = pl.ds(i * 16, 16)
        idx = src_v[sl]
        d = dst_v[sl]
        vals = plsc.load_gather(u_v, [idx])
        plsc.addupdate_scatter(acc_v, [d], vals)

    pltpu.sync_copy(acc_v, out.at[pl.ds(w * N_PAD, N_PAD)])


def _tc_final_body(acc2_ref, u_ref, dinv_ref, b2_ref, out_ref):
    t = jnp.sum(acc2_ref[...], axis=0) + u_ref[...]
    out_ref[...] = jax.nn.sigmoid(dinv_ref[...] * t + b2_ref[0, 0])


def _tc_final(acc2p, u2d, dinv2d, b2r):
    blk = 16
    return pl.pallas_call(
        _tc_final_body,
        grid=(N_PAD // 128 // blk,),
        in_specs=[
            pl.BlockSpec((NW, blk, 128), lambda i: (0, i, 0)),
            pl.BlockSpec((blk, 128), lambda i: (i, 0)),
            pl.BlockSpec((blk, 128), lambda i: (i, 0)),
            pl.BlockSpec((1, 1), lambda i: (0, 0), memory_space=pltpu.SMEM),
        ],
        out_specs=pl.BlockSpec((blk, 128), lambda i: (i, 0)),
        out_shape=jax.ShapeDtypeStruct((N_PAD // 128, 128), jnp.float32),
    )(acc2p, u2d, dinv2d, b2r)


@jax.jit
def kernel(x, edge_index, W1, b1, W2, b2):
    ei = edge_index.astype(jnp.int32)
    src = ei[0]
    dst = ei[1]
    npad = E_PAD - N_EDGES
    src_pade = jnp.concatenate([src, jnp.zeros((npad,), jnp.int32)])
    dst_pade = jnp.concatenate([dst, jnp.full((npad,), N_PAD - 1, jnp.int32)])
    zeros320 = jnp.zeros((NHALF // NS, HALF), jnp.float32)
    b1r = b1.reshape(1, HID_DIM)
    b2r = b2.reshape(1, 1)

    cnt_flat = _sc_count(dst_pade)
    dinv2d = _tc_dinv(cnt_flat.reshape(NW, N_PAD // 128, 128))
    dinv_col = dinv2d.reshape(N_PAD, 1)[:N_NODES]
    dinv16, y0, y1 = _tc_prep(dinv_col, x)
    acc0, acc1 = _sc_rowprop(src, dst, y0, y1, zeros320)
    u16 = _tc_mm(acc0, acc1, y0, y1, dinv16, W1, b1r, W2)

    pad_n = N_PAD - N_NODES
    u_flat = jnp.concatenate([u16[:, 0], jnp.zeros((pad_n,), jnp.float32)])
    u2d = u_flat.reshape(-1, 128)

    acc2_flat = _sc_scalarprop(src_pade, dst_pade, u_flat)
    acc2p = acc2_flat.reshape(NW, N_PAD // 128, 128)
    out2d = _tc_final(acc2p, u2d, dinv2d, b2r)
    return out2d.reshape(N_PAD, 1)[:N_NODES]


# speedup vs baseline: 15.9832x; 1.0343x over previous
import functools

import jax
import jax.numpy as jnp
from jax import lax
from jax.experimental import pallas as pl
from jax.experimental.pallas import tpu as pltpu
from jax.experimental.pallas import tpu_sc as plsc

N_NODES = 10000
N_EDGES = 160000
IN_DIM = 256
HID_DIM = 512

NC = 2
NS = 16
NW = NC * NS
N_PAD = 10240
RPT = N_PAD // NS

E_PAD = 160256
EPW = E_PAD // NW
EVECS = EPW // 16

HALF = IN_DIM // 2

_mesh = plsc.VectorSubcoreMesh(core_axis_name="c", subcore_axis_name="s")


@functools.partial(
    pl.kernel,
    out_type=jax.ShapeDtypeStruct((NW * N_PAD,), jnp.float32),
    mesh=_mesh,
    scratch_types=[
        pltpu.VMEM((EPW,), jnp.int32),
        pltpu.VMEM((N_PAD,), jnp.float32),
    ],
    compiler_params=pltpu.CompilerParams(needs_layout_passes=False),
)
def _sc_count(dst_flat, out, dst_v, acc_v):
    c = lax.axis_index("c")
    s = lax.axis_index("s")
    w = c * NS + s
    pltpu.sync_copy(dst_flat.at[pl.ds(w * EPW, EPW)], dst_v)

    zeros = jnp.zeros((16,), jnp.float32)
    ones = jnp.ones((16,), jnp.float32)

    @pl.loop(0, N_PAD // 16)
    def _(j):
        acc_v[pl.ds(j * 16, 16)] = zeros

    @pl.loop(0, EVECS)
    def _(i):
        d = dst_v[pl.ds(i * 16, 16)]
        plsc.addupdate_scatter(acc_v, [d], ones)

    pltpu.sync_copy(acc_v, out.at[pl.ds(w * N_PAD, N_PAD)])


def _tc_dinv_body(cnt_ref, dinv_ref):
    deg = jnp.sum(cnt_ref[...], axis=0) + 1.0
    dinv_ref[...] = lax.rsqrt(deg)


def _tc_dinv(cntp):
    blk = 16
    return pl.pallas_call(
        _tc_dinv_body,
        grid=(N_PAD // 128 // blk,),
        in_specs=[pl.BlockSpec((NW, blk, 128), lambda i: (0, i, 0))],
        out_specs=pl.BlockSpec((blk, 128), lambda i: (i, 0)),
        out_shape=jax.ShapeDtypeStruct((N_PAD // 128, 128), jnp.float32),
    )(cntp)


def _tc_prep_body(dinv_col_ref, x_ref, dinv_ref, y0_ref, y1_ref):
    dinv = dinv_col_ref[...]
    y = x_ref[...] * dinv
    y0_ref[...] = y[:, :HALF]
    y1_ref[...] = y[:, HALF:]
    dinv_ref[...] = jnp.broadcast_to(dinv, dinv_ref.shape)


def _tc_prep(dinv_col, x):
    blk = 1000
    yspec = pl.BlockSpec((blk, HALF), lambda i: (i, 0))
    yshape = jax.ShapeDtypeStruct((N_NODES, HALF), jnp.float32)
    return pl.pallas_call(
        _tc_prep_body,
        grid=(N_NODES // blk,),
        in_specs=[
            pl.BlockSpec((blk, 1), lambda i: (i, 0)),
            pl.BlockSpec((blk, IN_DIM), lambda i: (i, 0)),
        ],
        out_specs=[pl.BlockSpec((blk, 16), lambda i: (i, 0)), yspec, yspec],
        out_shape=[jax.ShapeDtypeStruct((N_NODES, 16), jnp.float32),
                   yshape, yshape],
    )(dinv_col, x)


NHALF = N_PAD // 2
RPC = NHALF // NS
EPT = N_EDGES // NS
CVECS = EPT // 16
RB = 128
CCAP = EPT + 240


@functools.partial(
    pl.kernel,
    out_type=[jax.ShapeDtypeStruct((N_PAD, HALF), jnp.float32)] * 2,
    mesh=_mesh,
    scratch_types=[
        pltpu.VMEM((EPT,), jnp.int32),
        pltpu.VMEM((EPT,), jnp.int32),
        pltpu.VMEM((CCAP,), jnp.int32),
        pltpu.VMEM((CCAP,), jnp.int32),
        pltpu.VMEM((RB,), jnp.int32),
        pltpu.VMEM((RB,), jnp.int32),
        pltpu.VMEM((RB,), jnp.int32),
        pltpu.VMEM((RB,), jnp.int32),
        pltpu.VMEM((RB, HALF), jnp.float32),
        pltpu.VMEM((RB, HALF), jnp.float32),
        pltpu.VMEM_SHARED((NHALF + 8, HALF), jnp.float32),
        pltpu.SemaphoreType.DMA,
        pltpu.SemaphoreType.DMA,
    ],
    compiler_params=pltpu.CompilerParams(needs_layout_passes=False),
)
def _sc_rowprop(src_h, dst_h, y0, y1, z320, out0, out1,
                src_v, dst_v, src_c, dst_c, src_sa, dst_sa, src_sb, dst_sb,
                gbufa, gbufb, acc, sema, semb):
    c = lax.axis_index("c")
    s = lax.axis_index("s")
    lo = c * NHALF
    pltpu.sync_copy(src_h.at[pl.ds(s * EPT, EPT)], src_v)
    pltpu.sync_copy(dst_h.at[pl.ds(s * EPT, EPT)], dst_v)

    @pl.loop(0, CVECS, init_carry=jnp.int32(0))
    def cnt(i, n):
        sl = pl.ds(i * 16, 16)
        sv = src_v[sl]
        dl = dst_v[sl] - lo
        m = (dl >= 0) & (dl < NHALF)
        plsc.store_compressed(src_c.at[pl.ds(n, 16)], sv, mask=m)
        plsc.store_compressed(dst_c.at[pl.ds(n, 16)], dl, mask=m)
        return n + jnp.sum(m.astype(jnp.int32))

    sent_src = jnp.zeros((16,), jnp.int32)
    sent_dst = jnp.full((16,), NHALF, jnp.int32)

    @pl.loop(0, RB // 16)
    def _(k):
        src_c[pl.ds(cnt + k * 16, 16)] = sent_src
        dst_c[pl.ds(cnt + k * 16, 16)] = sent_dst

    nb = (cnt + (RB - 1)) // RB

    def stage(k, src_st, dst_st):
        @pl.loop(0, RB // 16)
        def _(j):
            jl = pl.ds(j * 16, 16)
            src_st[jl] = src_c[pl.ds(k * RB + j * 16, 16)]
            dst_st[jl] = dst_c[pl.ds(k * RB + j * 16, 16)]

    def run(tbl, out):
        @pl.when(nb > 0)
        def _():
            stage(0, src_sa, dst_sa)
            pltpu.async_copy(tbl.at[src_sa], gbufa, sema)

        @pl.loop(0, nb, step=2)
        def _(k):
            pltpu.make_async_copy(tbl.at[src_sa], gbufa, sema).wait()

            @pl.when(k + 1 < nb)
            def _():
                stage(k + 1, src_sb, dst_sb)
                pltpu.async_copy(tbl.at[src_sb], gbufb, semb)

            pltpu.sync_copy(gbufa, acc.at[dst_sa], add=True)

            @pl.when(k + 1 < nb)
            def _():
                pltpu.make_async_copy(tbl.at[src_sb], gbufb, semb).wait()

                @pl.when(k + 2 < nb)
                def _():
                    stage(k + 2, src_sa, dst_sa)
                    pltpu.async_copy(tbl.at[src_sa], gbufa, sema)

                pltpu.sync_copy(gbufb, acc.at[dst_sb], add=True)

        plsc.subcore_barrier()
        pltpu.sync_copy(acc.at[pl.ds(s * RPC, RPC)],
                        out.at[pl.ds(lo + s * RPC, RPC)])

    pltpu.sync_copy(z320, acc.at[pl.ds(s * RPC, RPC)])
    plsc.subcore_barrier()
    run(y0, out0)
    pltpu.sync_copy(z320, acc.at[pl.ds(s * RPC, RPC)])
    plsc.subcore_barrier()
    run(y1, out1)


def _tc_mm_body(a0_ref, a1_ref, y0_ref, y1_ref, dinv_ref, w1_ref, b1_ref,
                w2_ref, u_ref):
    d = dinv_ref[:, 0:1]
    p0 = d * (a0_ref[...] + y0_ref[...])
    p1 = d * (a1_ref[...] + y1_ref[...])
    h = jnp.dot(p0, w1_ref[:HALF, :], preferred_element_type=jnp.float32)
    h = h + jnp.dot(p1, w1_ref[HALF:, :], preferred_element_type=jnp.float32)
    h = jnp.maximum(h + b1_ref[...], 0.0)
    z = jnp.dot(h, w2_ref[...], preferred_element_type=jnp.float32)
    u_ref[...] = jnp.broadcast_to(z * d, u_ref.shape)


def _tc_mm(acc0, acc1, y0, y1, dinv16, W1, b1r, W2):
    blk = 400
    hspec = pl.BlockSpec((blk, HALF), lambda i: (i, 0))
    return pl.pallas_call(
        _tc_mm_body,
        grid=(N_NODES // blk,),
        in_specs=[hspec, hspec, hspec, hspec,
                  pl.BlockSpec((blk, 16), lambda i: (i, 0)),
                  pl.BlockSpec((IN_DIM, HID_DIM), lambda i: (0, 0)),
                  pl.BlockSpec((1, HID_DIM), lambda i: (0, 0)),
                  pl.BlockSpec((HID_DIM, 1), lambda i: (0, 0))],
        out_specs=pl.BlockSpec((blk, 16), lambda i: (i, 0)),
        out_shape=jax.ShapeDtypeStruct((N_NODES, 16), jnp.float32),
    )(acc0, acc1, y0, y1, dinv16, W1, b1r, W2)


@functools.partial(
    pl.kernel,
    out_type=jax.ShapeDtypeStruct((NW * N_PAD,), jnp.float32),
    mesh=_mesh,
    scratch_types=[
        pltpu.VMEM((EPW,), jnp.int32),
        pltpu.VMEM((EPW,), jnp.int32),
        pltpu.VMEM((N_PAD,), jnp.float32),
        pltpu.VMEM((N_PAD,), jnp.float32),
    ],
    compiler_params=pltpu.CompilerParams(needs_layout_passes=False),
)
def _sc_scalarprop(src_flat, dst_flat, u_flat, out, src_v, dst_v, u_v, acc_v):
    c = lax.axis_index("c")
    s = lax.axis_index("s")
    w = c * NS + s
    pltpu.sync_copy(src_flat.at[pl.ds(w * EPW, EPW)], src_v)
    pltpu.sync_copy(dst_flat.at[pl.ds(w * EPW, EPW)], dst_v)
    pltpu.sync_copy(u_flat, u_v)

    zeros = jnp.zeros((16,), jnp.float32)

    @pl.loop(0, N_PAD // 16)
    def _(j):
        acc_v[pl.ds(j * 16, 16)] = zeros

    @pl.loop(0, EVECS)
    def _(i):
        sl = pl.ds(i * 16, 16)
        idx = src_v[sl]
        d = dst_v[sl]
        vals = plsc.load_gather(u_v, [idx])
        plsc.addupdate_scatter(acc_v, [d], vals)

    pltpu.sync_copy(acc_v, out.at[pl.ds(w * N_PAD, N_PAD)])


def _tc_final_body(acc2_ref, u_ref, dinv_ref, b2_ref, out_ref):
    t = jnp.sum(acc2_ref[...], axis=0) + u_ref[...]
    out_ref[...] = jax.nn.sigmoid(dinv_ref[...] * t + b2_ref[0, 0])


def _tc_final(acc2p, u2d, dinv2d, b2r):
    blk = 16
    return pl.pallas_call(
        _tc_final_body,
        grid=(N_PAD // 128 // blk,),
        in_specs=[
            pl.BlockSpec((NW, blk, 128), lambda i: (0, i, 0)),
            pl.BlockSpec((blk, 128), lambda i: (i, 0)),
            pl.BlockSpec((blk, 128), lambda i: (i, 0)),
            pl.BlockSpec((1, 1), lambda i: (0, 0), memory_space=pltpu.SMEM),
        ],
        out_specs=pl.BlockSpec((blk, 128), lambda i: (i, 0)),
        out_shape=jax.ShapeDtypeStruct((N_PAD // 128, 128), jnp.float32),
    )(acc2p, u2d, dinv2d, b2r)


@jax.jit
def kernel(x, edge_index, W1, b1, W2, b2):
    ei = edge_index.astype(jnp.int32)
    src = ei[0]
    dst = ei[1]
    npad = E_PAD - N_EDGES
    src_pade = jnp.concatenate([src, jnp.zeros((npad,), jnp.int32)])
    dst_pade = jnp.concatenate([dst, jnp.full((npad,), N_PAD - 1, jnp.int32)])
    zeros320 = jnp.zeros((NHALF // NS, HALF), jnp.float32)
    b1r = b1.reshape(1, HID_DIM)
    b2r = b2.reshape(1, 1)

    cnt_flat = _sc_count(dst_pade)
    dinv2d = _tc_dinv(cnt_flat.reshape(NW, N_PAD // 128, 128))
    dinv_col = dinv2d.reshape(N_PAD, 1)[:N_NODES]
    dinv16, y0, y1 = _tc_prep(dinv_col, x)
    acc0, acc1 = _sc_rowprop(src, dst, y0, y1, zeros320)
    u16 = _tc_mm(acc0, acc1, y0, y1, dinv16, W1, b1r, W2)

    pad_n = N_PAD - N_NODES
    u_flat = jnp.concatenate([u16[:, 0], jnp.zeros((pad_n,), jnp.float32)])
    u2d = u_flat.reshape(-1, 128)

    acc2_flat = _sc_scalarprop(src_pade, dst_pade, u_flat)
    acc2p = acc2_flat.reshape(NW, N_PAD // 128, 128)
    out2d = _tc_final(acc2p, u2d, dinv2d, b2r)
    return out2d.reshape(N_PAD, 1)[:N_NODES]
